# baseline (device time: 11115 ns/iter reference)
import jax
import jax.numpy as jnp
from jax import lax
from jax.experimental import pallas as pl
from jax.experimental.pallas import tpu as pltpu

T = 256
D = 512
V_LOCAL = 4096
CHUNK = 512
N_CHUNKS = V_LOCAL // CHUNK


def kernel(x, W, labels):
    def body(x_hbm, w_ref, labels_hbm, out_ref,
             x_scr, lab_scr, stats1_ref, stats2_ref,
             recv1_ref, recv2_ref, x_sem, lab_sem, sems):
        my_x = lax.axis_index("x")
        my_y = lax.axis_index("y")
        nbr = (my_x, 1 - my_y)

        x_dma = pltpu.make_async_copy(x_hbm, x_scr, x_sem)
        lab_dma = pltpu.make_async_copy(labels_hbm, lab_scr, lab_sem)
        x_dma.start()
        lab_dma.start()

        barrier_sem = pltpu.get_barrier_semaphore()
        pl.semaphore_signal(barrier_sem, inc=1, device_id=nbr,
                            device_id_type=pl.DeviceIdType.MESH)

        x_dma.wait()
        lab_dma.wait()
        xb = x_scr[:, :].astype(jnp.bfloat16)
        idx = lab_scr[:, :] - my_y * V_LOCAL

        def half(c0, c1):
            s = jnp.zeros((1, T), jnp.float32)
            ll = jnp.zeros((1, T), jnp.float32)
            for c in range(c0, c1):
                wb = w_ref[:, c * CHUNK:(c + 1) * CHUNK].astype(jnp.bfloat16)
                lgT = lax.dot_general(wb, xb, (((0,), (1,)), ((), ())),
                                      preferred_element_type=jnp.float32)
                s = s + jnp.sum(jnp.exp(lgT), axis=0, keepdims=True)
                rows = (c * CHUNK
                        + lax.broadcasted_iota(jnp.int32, (CHUNK, T), 0))
                ll = ll + jnp.sum(jnp.where(rows == idx, lgT, 0.0),
                                  axis=0, keepdims=True)
            return s, ll

        s1, ll1 = half(0, N_CHUNKS // 2)
        stats1_ref[0:1, :] = s1
        stats1_ref[1:2, :] = ll1
        pl.semaphore_wait(barrier_sem, 1)
        rdma1 = pltpu.make_async_remote_copy(
            src_ref=stats1_ref, dst_ref=recv1_ref,
            send_sem=sems.at[0], recv_sem=sems.at[1],
            device_id=nbr, device_id_type=pl.DeviceIdType.MESH,
        )
        rdma1.start()

        s2, ll2 = half(N_CHUNKS // 2, N_CHUNKS)
        stats2_ref[0:1, :] = s2
        stats2_ref[1:2, :] = ll2
        rdma2 = pltpu.make_async_remote_copy(
            src_ref=stats2_ref, dst_ref=recv2_ref,
            send_sem=sems.at[2], recv_sem=sems.at[3],
            device_id=nbr, device_id_type=pl.DeviceIdType.MESH,
        )
        rdma2.start()
        rdma1.wait()
        rdma2.wait()
        s = s1 + s2 + recv1_ref[0:1, :] + recv2_ref[0:1, :]
        ll = ll1 + ll2 + recv1_ref[1:2, :] + recv2_ref[1:2, :]
        out_ref[:, :] = jnp.log(s) - ll

    out = pl.pallas_call(
        body,
        out_shape=jax.ShapeDtypeStruct((1, T), jnp.float32),
        in_specs=[
            pl.BlockSpec(memory_space=pl.ANY),
            pl.BlockSpec(memory_space=pltpu.VMEM),
            pl.BlockSpec(memory_space=pl.ANY),
        ],
        out_specs=pl.BlockSpec(memory_space=pltpu.VMEM),
        scratch_shapes=[
            pltpu.VMEM((T, D), jnp.float32),
            pltpu.VMEM((1, T), jnp.int32),
            pltpu.VMEM((2, T), jnp.float32),
            pltpu.VMEM((2, T), jnp.float32),
            pltpu.VMEM((2, T), jnp.float32),
            pltpu.VMEM((2, T), jnp.float32),
            pltpu.SemaphoreType.DMA,
            pltpu.SemaphoreType.DMA,
            pltpu.SemaphoreType.DMA((4,)),
        ],
        compiler_params=pltpu.CompilerParams(collective_id=0),
    )(
        pltpu.with_memory_space_constraint(x, pltpu.MemorySpace.HBM),
        W,
        pltpu.with_memory_space_constraint(
            labels.reshape(1, T), pltpu.MemorySpace.HBM),
    )
    return out.reshape(T)


# device time: 10689 ns/iter; 1.0399x vs baseline; 1.0399x over previous
import jax
import jax.numpy as jnp
from jax import lax
from jax.experimental import pallas as pl
from jax.experimental.pallas import tpu as pltpu

T = 256
D = 512
V_LOCAL = 4096
CHUNK = 512
N_CHUNKS = V_LOCAL // CHUNK


def kernel(x, W, labels):
    def body(x_hbm, w_ref, labels_hbm, out_ref,
             x_scr, lab_scr, stats_ref, recv_ref,
             x_sem, lab_sem, send_sem, recv_sem):
        my_x = lax.axis_index("x")
        my_y = lax.axis_index("y")
        nbr = (my_x, 1 - my_y)

        x_dma = pltpu.make_async_copy(x_hbm, x_scr, x_sem)
        lab_dma = pltpu.make_async_copy(labels_hbm, lab_scr, lab_sem)
        x_dma.start()
        lab_dma.start()

        barrier_sem = pltpu.get_barrier_semaphore()
        pl.semaphore_signal(barrier_sem, inc=1, device_id=nbr,
                            device_id_type=pl.DeviceIdType.MESH)

        x_dma.wait()
        lab_dma.wait()
        xb = x_scr[:, :].astype(jnp.bfloat16)
        idx = lab_scr[:, :] - my_y * V_LOCAL
        s = jnp.zeros((1, T), jnp.float32)
        ll = jnp.zeros((1, T), jnp.float32)
        for c in range(N_CHUNKS):
            wb = w_ref[:, c * CHUNK:(c + 1) * CHUNK].astype(jnp.bfloat16)
            lgT = lax.dot_general(wb, xb, (((0,), (1,)), ((), ())),
                                  preferred_element_type=jnp.float32)
            s = s + jnp.sum(jnp.exp(lgT), axis=0, keepdims=True)
            rows = c * CHUNK + lax.broadcasted_iota(jnp.int32, (CHUNK, T), 0)
            ll = ll + jnp.sum(jnp.where(rows == idx, lgT, 0.0),
                              axis=0, keepdims=True)

        stats_ref[0:1, :] = s
        stats_ref[1:2, :] = ll
        pl.semaphore_wait(barrier_sem, 1)
        rdma = pltpu.make_async_remote_copy(
            src_ref=stats_ref, dst_ref=recv_ref,
            send_sem=send_sem, recv_sem=recv_sem,
            device_id=nbr, device_id_type=pl.DeviceIdType.MESH,
        )
        rdma.start()
        rdma.wait()
        out_ref[:, :] = (jnp.log(s + recv_ref[0:1, :])
                         - (ll + recv_ref[1:2, :]))

    out = pl.pallas_call(
        body,
        out_shape=jax.ShapeDtypeStruct((1, T), jnp.float32),
        in_specs=[
            pl.BlockSpec(memory_space=pl.ANY),
            pl.BlockSpec(memory_space=pltpu.VMEM),
            pl.BlockSpec(memory_space=pl.ANY),
        ],
        out_specs=pl.BlockSpec(memory_space=pltpu.VMEM),
        scratch_shapes=[
            pltpu.VMEM((T, D), jnp.float32),
            pltpu.VMEM((1, T), jnp.int32),
            pltpu.VMEM((2, T), jnp.float32),
            pltpu.VMEM((2, T), jnp.float32),
            pltpu.SemaphoreType.DMA,
            pltpu.SemaphoreType.DMA,
            pltpu.SemaphoreType.DMA,
            pltpu.SemaphoreType.DMA,
        ],
        compiler_params=pltpu.CompilerParams(collective_id=0),
    )(
        pltpu.with_memory_space_constraint(x, pltpu.MemorySpace.HBM),
        W,
        pltpu.with_memory_space_constraint(
            labels.reshape(1, T), pltpu.MemorySpace.HBM),
    )
    return out.reshape(T)
